# baseline (speedup 1.0000x reference)
import jax
import jax.numpy as jnp
from jax.experimental import pallas as pl
from jax.experimental.pallas import tpu as pltpu

LATENT_DIM = 64
HIDDEN = 128
NUM_EMB = 1024
BETA = 0.25

_P2 = 57
_NP2 = 3328
_ZL = 3192


def _enc2_vq_kernel(planes_ref, w_ref, b_ref, cb_ref, cb2_ref, mask_ref,
                    z_ref, loss_ref, acc_ref):
    bidx = pl.program_id(0)
    z = jnp.broadcast_to(b_ref[...], (LATENT_DIM, _ZL))
    for pi in range(4):
        y = jax.lax.dot_general(
            w_ref[pi], planes_ref[0, pi], (((1,), (0,)), ((), ())),
            preferred_element_type=jnp.float32)
        for a in range(2):
            for b in range(2):
                ck = a * 2 + b
                off = a * _P2 + b
                z = z + y[ck * LATENT_DIM:(ck + 1) * LATENT_DIM,
                          off:off + _ZL]
    z_ref[0] = z
    xc = jax.lax.dot_general(cb_ref[...], z, (((1,), (0,)), ((), ())),
                             preferred_element_type=jnp.float32)
    d = cb2_ref[...] - 2.0 * xc
    m = jnp.min(d, axis=0)
    z2 = jnp.sum(z * z, axis=0)
    partial = jnp.sum((m + z2) * mask_ref[0])

    @pl.when(bidx == 0)
    def _init():
        acc_ref[0, 0] = 0.0

    acc_ref[0, 0] += partial

    @pl.when(bidx == pl.num_programs(0) - 1)
    def _fin():
        loss_ref[0, 0] = acc_ref[0, 0]


def _enc2_vq(h, w, bias, codebook):
    B = h.shape[0]
    hp = jnp.pad(h, ((0, 0), (0, 0), (1, 1), (1, 1)))
    planes = jnp.stack(
        [hp[:, :, p::2, q::2].reshape(B, HIDDEN, _P2 * _P2)
         for p in range(2) for q in range(2)], axis=1)
    planes = jnp.pad(planes, ((0, 0), (0, 0), (0, 0),
                              (0, _NP2 - _P2 * _P2)))
    wt = w.transpose(2, 3, 0, 1)
    wstack = jnp.stack(
        [jnp.concatenate([wt[2 * a + p, 2 * b + q]
                          for a in range(2) for b in range(2)], axis=0)
         for p in range(2) for q in range(2)], axis=0)
    col = jax.lax.broadcasted_iota(jnp.int32, (1, _ZL), 1)
    mask = jnp.where(col % _P2 == _P2 - 1, 0.0, 1.0).astype(jnp.float32)
    cb2 = jnp.sum(codebook * codebook, axis=1)[:, None]
    z_flat, total = pl.pallas_call(
        _enc2_vq_kernel,
        grid=(B,),
        in_specs=[
            pl.BlockSpec((1, 4, HIDDEN, _NP2), lambda b: (b, 0, 0, 0)),
            pl.BlockSpec((4, 4 * LATENT_DIM, HIDDEN), lambda b: (0, 0, 0)),
            pl.BlockSpec((LATENT_DIM, 1), lambda b: (0, 0)),
            pl.BlockSpec((NUM_EMB, LATENT_DIM), lambda b: (0, 0)),
            pl.BlockSpec((NUM_EMB, 1), lambda b: (0, 0)),
            pl.BlockSpec((1, _ZL), lambda b: (0, 0)),
        ],
        out_specs=[
            pl.BlockSpec((1, LATENT_DIM, _ZL), lambda b: (b, 0, 0)),
            pl.BlockSpec((1, 1), lambda b: (0, 0), memory_space=pltpu.SMEM),
        ],
        out_shape=[
            jax.ShapeDtypeStruct((B, LATENT_DIM, _ZL), jnp.float32),
            jax.ShapeDtypeStruct((1, 1), jnp.float32),
        ],
        scratch_shapes=[pltpu.SMEM((1, 1), jnp.float32)],
    )(planes, wstack, bias[:, None], codebook, cb2, mask)
    return z_flat, total[0, 0]


def _conv(x, w, b, stride):
    y = jax.lax.conv_general_dilated(
        x, w, window_strides=(stride, stride), padding=((1, 1), (1, 1)),
        dimension_numbers=('NCHW', 'OIHW', 'NCHW'))
    return y + b[None, :, None, None]


def _conv_transpose(x, w, b):
    w_flip = w[:, :, ::-1, ::-1]
    y = jax.lax.conv_general_dilated(
        x, w_flip, window_strides=(1, 1), padding=((2, 2), (2, 2)),
        lhs_dilation=(2, 2), dimension_numbers=('NCHW', 'OIHW', 'NCHW'))
    return y + b[None, :, None, None]


def kernel(img, enc_w1, enc_b1, enc_w2, enc_b2, codebook, dec_w1, dec_b1, dec_w2, dec_b2):
    h = jax.nn.relu(_conv(img, enc_w1, enc_b1, 2))
    z_flat, total = _enc2_vq(h, enc_w2, enc_b2, codebook)
    B = img.shape[0]
    loss = (1.0 + BETA) * total / (B * LATENT_DIM * 56 * 56)
    z = z_flat.reshape(B, LATENT_DIM, 56, _P2)[:, :, :, :56]
    hd = jax.nn.relu(_conv_transpose(z, dec_w1, dec_b1))
    out = _conv_transpose(hd, dec_w2, dec_b2)
    return (out, loss)
